# Initial kernel scaffold; baseline (speedup 1.0000x reference)
import jax
import jax.numpy as jnp
from jax.experimental import pallas as pl


def kernel(x, batch, W1, b1, W2, b2):
    raise NotImplementedError("write your pallas kernel here")


# speedup vs baseline: 3.3178x; 3.3178x over previous
import jax
import jax.numpy as jnp
from jax import lax
from jax.experimental import pallas as pl
from jax.experimental.pallas import tpu as pltpu

N = 100000
D = 128
H = 64
S = 512
R1 = 2000
R2 = 2000
G1 = N // R1
G2 = N // R2


def _gate_kernel(x_ref, w1_ref, b1_ref, w2_ref, logits_ref, m_ref, z_ref):
    i = pl.program_id(0)
    h = jnp.dot(x_ref[...], w1_ref[...], preferred_element_type=jnp.float32)
    h = jnp.maximum(h + b1_ref[...], 0.0)
    g = jnp.sum(h * w2_ref[...], axis=1)
    logits_ref[0, 0, :] = g

    @pl.when(i == 0)
    def _():
        m_ref[0, 0] = -jnp.inf
        z_ref[0, 0] = 0.0

    m_old = m_ref[0, 0]
    m_new = jnp.maximum(m_old, jnp.max(g))
    z_ref[0, 0] = z_ref[0, 0] * jnp.exp(m_old - m_new) + jnp.sum(jnp.exp(g - m_new))
    m_ref[0, 0] = m_new


def _pool_kernel(x_ref, logits_ref, ids_ref, m_ref, z_ref, out_ref):
    i = pl.program_id(0)
    g = logits_ref[0, 0, :]
    w = jnp.exp(g - m_ref[0, 0]) * (1.0 / z_ref[0, 0])
    ids = ids_ref[0, 0, :]
    seg = lax.broadcasted_iota(jnp.int32, (S, R2), 0)
    ohw = jnp.where(ids[None, :] == seg, w[None, :], 0.0)
    contrib = jnp.dot(ohw, x_ref[...], preferred_element_type=jnp.float32)

    @pl.when(i == 0)
    def _():
        out_ref[...] = jnp.zeros_like(out_ref)

    out_ref[...] += contrib


def kernel(x, batch, W1, b1, W2, b2):
    del b2
    ids3 = batch.astype(jnp.int32).reshape(G2, 1, R2)
    b1r = b1.reshape(1, H)
    w2r = W2.reshape(1, H)
    logits, m, z = pl.pallas_call(
        _gate_kernel,
        grid=(G1,),
        in_specs=[
            pl.BlockSpec((R1, D), lambda i: (i, 0)),
            pl.BlockSpec((D, H), lambda i: (0, 0)),
            pl.BlockSpec((1, H), lambda i: (0, 0)),
            pl.BlockSpec((1, H), lambda i: (0, 0)),
        ],
        out_specs=[
            pl.BlockSpec((1, 1, R1), lambda i: (i, 0, 0)),
            pl.BlockSpec(memory_space=pltpu.SMEM),
            pl.BlockSpec(memory_space=pltpu.SMEM),
        ],
        out_shape=[
            jax.ShapeDtypeStruct((G1, 1, R1), jnp.float32),
            jax.ShapeDtypeStruct((1, 1), jnp.float32),
            jax.ShapeDtypeStruct((1, 1), jnp.float32),
        ],
    )(x, W1, b1r, w2r)
    out = pl.pallas_call(
        _pool_kernel,
        grid=(G2,),
        in_specs=[
            pl.BlockSpec((R2, D), lambda i: (i, 0)),
            pl.BlockSpec((1, 1, R2), lambda i: (i, 0, 0)),
            pl.BlockSpec((1, 1, R2), lambda i: (i, 0, 0)),
            pl.BlockSpec(memory_space=pltpu.SMEM),
            pl.BlockSpec(memory_space=pltpu.SMEM),
        ],
        out_specs=pl.BlockSpec((S, D), lambda i: (0, 0)),
        out_shape=jax.ShapeDtypeStruct((S, D), jnp.float32),
    )(x, logits, ids3, m, z)
    return out


# speedup vs baseline: 3.3545x; 1.0110x over previous
import jax
import jax.numpy as jnp
from jax import lax
from jax.experimental import pallas as pl
from jax.experimental.pallas import tpu as pltpu

N = 100000
D = 128
H = 64
S = 512
R1 = 2000
R2 = 2000
G1 = N // R1
G2 = N // R2


def _gate_kernel(x_ref, w1_ref, b1_ref, w2_ref, logits_ref, m_ref, z_ref):
    i = pl.program_id(0)
    h = jnp.dot(x_ref[...], w1_ref[...], preferred_element_type=jnp.float32)
    h = jnp.maximum(h + b1_ref[...], 0.0)
    g = jnp.dot(h, w2_ref[...], preferred_element_type=jnp.float32)
    logits_ref[...] = g

    @pl.when(i == 0)
    def _():
        m_ref[0, 0] = -jnp.inf
        z_ref[0, 0] = 0.0

    m_old = m_ref[0, 0]
    m_new = jnp.maximum(m_old, jnp.max(g))
    z_ref[0, 0] = z_ref[0, 0] * jnp.exp(m_old - m_new) + jnp.sum(jnp.exp(g - m_new))
    m_ref[0, 0] = m_new


def _pool_kernel(x_ref, logits_ref, ids_ref, m_ref, z_ref, out_ref):
    i = pl.program_id(0)
    w = jnp.exp(logits_ref[...] - m_ref[0, 0]) * (1.0 / z_ref[0, 0])
    y = (x_ref[...] * w).astype(jnp.bfloat16)
    ids = ids_ref[0, 0, :]
    seg = lax.broadcasted_iota(jnp.int32, (S, R2), 0)
    onehot = (ids[None, :] == seg).astype(jnp.bfloat16)
    contrib = jnp.dot(onehot, y, preferred_element_type=jnp.float32)

    @pl.when(i == 0)
    def _():
        out_ref[...] = jnp.zeros_like(out_ref)

    out_ref[...] += contrib


def kernel(x, batch, W1, b1, W2, b2):
    del b2
    ids3 = batch.astype(jnp.int32).reshape(G2, 1, R2)
    b1r = b1.reshape(1, H)
    logits, m, z = pl.pallas_call(
        _gate_kernel,
        grid=(G1,),
        in_specs=[
            pl.BlockSpec((R1, D), lambda i: (i, 0)),
            pl.BlockSpec((D, H), lambda i: (0, 0)),
            pl.BlockSpec((1, H), lambda i: (0, 0)),
            pl.BlockSpec((H, 1), lambda i: (0, 0)),
        ],
        out_specs=[
            pl.BlockSpec((R1, 1), lambda i: (i, 0)),
            pl.BlockSpec(memory_space=pltpu.SMEM),
            pl.BlockSpec(memory_space=pltpu.SMEM),
        ],
        out_shape=[
            jax.ShapeDtypeStruct((N, 1), jnp.float32),
            jax.ShapeDtypeStruct((1, 1), jnp.float32),
            jax.ShapeDtypeStruct((1, 1), jnp.float32),
        ],
    )(x, W1, b1r, W2)
    out = pl.pallas_call(
        _pool_kernel,
        grid=(G2,),
        in_specs=[
            pl.BlockSpec((R2, D), lambda i: (i, 0)),
            pl.BlockSpec((R2, 1), lambda i: (i, 0)),
            pl.BlockSpec((1, 1, R2), lambda i: (i, 0, 0)),
            pl.BlockSpec(memory_space=pltpu.SMEM),
            pl.BlockSpec(memory_space=pltpu.SMEM),
        ],
        out_specs=pl.BlockSpec((S, D), lambda i: (0, 0)),
        out_shape=jax.ShapeDtypeStruct((S, D), jnp.float32),
    )(x, logits, ids3, m, z)
    return out


# speedup vs baseline: 4.3384x; 1.2933x over previous
import jax
import jax.numpy as jnp
from jax import lax
from jax.experimental import pallas as pl
from jax.experimental.pallas import tpu as pltpu

N = 100000
D = 128
H = 64
S = 512
SSUB = 64
R1 = 2000
R2 = 2000
G1 = N // R1
G2 = N // R2


def _gate_kernel(x_ref, w1_ref, b1_ref, w2_ref, logits_ref, xb_ref, m_ref, z_ref):
    i = pl.program_id(0)
    xv = x_ref[...]
    xb_ref[...] = xv.astype(jnp.bfloat16)
    h = jnp.dot(xv, w1_ref[...], preferred_element_type=jnp.float32)
    h = jnp.maximum(h + b1_ref[...], 0.0)
    g_col = jnp.dot(h, w2_ref[...], preferred_element_type=jnp.float32)
    g = jnp.transpose(g_col)
    logits_ref[0] = g

    @pl.when(i == 0)
    def _():
        m_ref[0, 0] = -jnp.inf
        z_ref[0, 0] = 0.0

    m_old = m_ref[0, 0]
    m_new = jnp.maximum(m_old, jnp.max(g))
    z_ref[0, 0] = z_ref[0, 0] * jnp.exp(m_old - m_new) + jnp.sum(jnp.exp(g - m_new))
    m_ref[0, 0] = m_new


def _pool_kernel(xb_ref, logits_ref, ids_ref, ids_s_ref, m_ref, z_ref, out_ref):
    i = pl.program_id(0)
    w = jnp.exp(logits_ref[0] - m_ref[0, 0]) * (1.0 / z_ref[0, 0])
    ids = ids_ref[0, 0, :]
    first = ids_s_ref[0, 0, 0]
    last = ids_s_ref[0, 0, R2 - 1]
    base = jnp.minimum((first // 8) * 8, S - SSUB)
    fits = (last - base) < SSUB

    @pl.when(i == 0)
    def _():
        out_ref[...] = jnp.zeros_like(out_ref)

    @pl.when(fits)
    def _():
        shifted = ids - base
        seg = lax.broadcasted_iota(jnp.int32, (SSUB, R2), 0)
        ohw = jnp.where(shifted[None, :] == seg, w, 0.0).astype(jnp.bfloat16)
        contrib = jnp.dot(ohw, xb_ref[...], preferred_element_type=jnp.float32)
        out_ref[pl.ds(base, SSUB), :] += contrib

    @pl.when(jnp.logical_not(fits))
    def _():
        seg = lax.broadcasted_iota(jnp.int32, (S, R2), 0)
        ohw = jnp.where(ids[None, :] == seg, w, 0.0).astype(jnp.bfloat16)
        contrib = jnp.dot(ohw, xb_ref[...], preferred_element_type=jnp.float32)
        out_ref[...] += contrib


def kernel(x, batch, W1, b1, W2, b2):
    del b2
    ids3 = batch.astype(jnp.int32).reshape(G2, 1, R2)
    b1r = b1.reshape(1, H)
    logits, xb, m, z = pl.pallas_call(
        _gate_kernel,
        grid=(G1,),
        in_specs=[
            pl.BlockSpec((R1, D), lambda i: (i, 0)),
            pl.BlockSpec((D, H), lambda i: (0, 0)),
            pl.BlockSpec((1, H), lambda i: (0, 0)),
            pl.BlockSpec((H, 1), lambda i: (0, 0)),
        ],
        out_specs=[
            pl.BlockSpec((1, 1, R1), lambda i: (i, 0, 0)),
            pl.BlockSpec((R1, D), lambda i: (i, 0)),
            pl.BlockSpec(memory_space=pltpu.SMEM),
            pl.BlockSpec(memory_space=pltpu.SMEM),
        ],
        out_shape=[
            jax.ShapeDtypeStruct((G1, 1, R1), jnp.float32),
            jax.ShapeDtypeStruct((N, D), jnp.bfloat16),
            jax.ShapeDtypeStruct((1, 1), jnp.float32),
            jax.ShapeDtypeStruct((1, 1), jnp.float32),
        ],
    )(x, W1, b1r, W2)
    out = pl.pallas_call(
        _pool_kernel,
        grid=(G2,),
        in_specs=[
            pl.BlockSpec((R2, D), lambda i: (i, 0)),
            pl.BlockSpec((1, 1, R2), lambda i: (i, 0, 0)),
            pl.BlockSpec((1, 1, R2), lambda i: (i, 0, 0)),
            pl.BlockSpec((1, 1, R2), lambda i: (i, 0, 0), memory_space=pltpu.SMEM),
            pl.BlockSpec(memory_space=pltpu.SMEM),
            pl.BlockSpec(memory_space=pltpu.SMEM),
        ],
        out_specs=pl.BlockSpec((S, D), lambda i: (0, 0)),
        out_shape=jax.ShapeDtypeStruct((S, D), jnp.float32),
    )(xb, logits, ids3, ids3, m, z)
    return out


# speedup vs baseline: 6.6008x; 1.5215x over previous
import jax
import jax.numpy as jnp
from jax import lax
from jax.experimental import pallas as pl
from jax.experimental.pallas import tpu as pltpu

N = 100000
D = 128
H = 64
S = 512
SSUB = 64
R = 2000
G = N // R


def _fused_kernel(x_ref, ids_ref, ids_s_ref, w1_ref, b1_ref, w2_ref,
                  out_ref, m_ref, z_ref):
    i = pl.program_id(0)
    xv = x_ref[...]
    h = jnp.dot(xv, w1_ref[...], preferred_element_type=jnp.float32)
    h = jnp.maximum(h + b1_ref[...], 0.0)
    g_col = jnp.dot(h, w2_ref[...], preferred_element_type=jnp.float32)
    g = jnp.transpose(g_col)

    @pl.when(i == 0)
    def _():
        m_ref[0, 0] = -jnp.inf
        z_ref[0, 0] = 0.0
        out_ref[...] = jnp.zeros_like(out_ref)

    m_old = m_ref[0, 0]
    m_new = jnp.maximum(m_old, jnp.max(g))
    m_ref[0, 0] = m_new
    scale = jnp.exp(m_old - m_new)

    @pl.when(jnp.logical_and(i > 0, scale < 1.0))
    def _():
        out_ref[...] *= scale

    e = jnp.exp(g - m_new)
    z_ref[0, 0] = z_ref[0, 0] * scale + jnp.sum(e)

    xb = xv.astype(jnp.bfloat16)
    ids = ids_ref[0, 0, :]
    first = ids_s_ref[0, 0, 0]
    last = ids_s_ref[0, 0, R - 1]
    base = jnp.minimum((first // 8) * 8, S - SSUB)
    fits = (last - base) < SSUB

    @pl.when(fits)
    def _():
        shifted = ids - base
        seg = lax.broadcasted_iota(jnp.int32, (SSUB, R), 0)
        ohw = jnp.where(shifted[None, :] == seg, e, 0.0).astype(jnp.bfloat16)
        contrib = jnp.dot(ohw, xb, preferred_element_type=jnp.float32)
        out_ref[pl.ds(base, SSUB), :] += contrib

    @pl.when(jnp.logical_not(fits))
    def _():
        seg = lax.broadcasted_iota(jnp.int32, (S, R), 0)
        ohw = jnp.where(ids[None, :] == seg, e, 0.0).astype(jnp.bfloat16)
        contrib = jnp.dot(ohw, xb, preferred_element_type=jnp.float32)
        out_ref[...] += contrib

    @pl.when(i == G - 1)
    def _():
        out_ref[...] *= (1.0 / z_ref[0, 0])


def kernel(x, batch, W1, b1, W2, b2):
    del b2
    ids3 = batch.astype(jnp.int32).reshape(G, 1, R)
    b1r = b1.reshape(1, H)
    out = pl.pallas_call(
        _fused_kernel,
        grid=(G,),
        in_specs=[
            pl.BlockSpec((R, D), lambda i: (i, 0)),
            pl.BlockSpec((1, 1, R), lambda i: (i, 0, 0)),
            pl.BlockSpec((1, 1, R), lambda i: (i, 0, 0), memory_space=pltpu.SMEM),
            pl.BlockSpec((D, H), lambda i: (0, 0)),
            pl.BlockSpec((1, H), lambda i: (0, 0)),
            pl.BlockSpec((H, 1), lambda i: (0, 0)),
        ],
        out_specs=pl.BlockSpec((S, D), lambda i: (0, 0)),
        out_shape=jax.ShapeDtypeStruct((S, D), jnp.float32),
        scratch_shapes=[
            pltpu.SMEM((1, 1), jnp.float32),
            pltpu.SMEM((1, 1), jnp.float32),
        ],
    )(x, ids3, ids3, W1, b1r, W2)
    return out


# speedup vs baseline: 9.6102x; 1.4559x over previous
import jax
import jax.numpy as jnp
from jax import lax
from jax.experimental import pallas as pl
from jax.experimental.pallas import tpu as pltpu

N = 100000
D = 128
H = 64
S = 512
SSUB = 64
R = 4000
G = N // R


def _fused_kernel(x_ref, ids_ref, ids_s_ref, w1_ref, b1_ref, w2_ref,
                  out_ref, m_ref, z_ref):
    i = pl.program_id(0)
    xv = x_ref[...]
    h = jnp.dot(xv, w1_ref[...], preferred_element_type=jnp.float32)
    h = jnp.maximum(h + b1_ref[...], 0.0)
    g_col = jnp.dot(h, w2_ref[...], preferred_element_type=jnp.float32)
    g = jnp.transpose(g_col)

    @pl.when(i == 0)
    def _():
        m_ref[0, 0] = -jnp.inf
        z_ref[0, 0] = 0.0
        out_ref[...] = jnp.zeros_like(out_ref)

    m_old = m_ref[0, 0]
    m_new = jnp.maximum(m_old, jnp.max(g))
    m_ref[0, 0] = m_new
    scale = jnp.exp(m_old - m_new)

    @pl.when(jnp.logical_and(i > 0, scale < 1.0))
    def _():
        out_ref[...] *= scale

    e = jnp.exp(g - m_new)
    z_ref[0, 0] = z_ref[0, 0] * scale + jnp.sum(e)

    xb = xv.astype(jnp.bfloat16)
    ids = ids_ref[0, 0, :]
    first = ids_s_ref[0, 0, 0]
    last = ids_s_ref[0, 0, R - 1]
    base = jnp.minimum((first // 8) * 8, S - SSUB)
    fits = (last - base) < SSUB

    @pl.when(fits)
    def _():
        shifted = ids - base
        seg = lax.broadcasted_iota(jnp.int32, (SSUB, R), 0)
        ohw = jnp.where(shifted[None, :] == seg, e, 0.0).astype(jnp.bfloat16)
        contrib = jnp.dot(ohw, xb, preferred_element_type=jnp.float32)
        out_ref[pl.ds(base, SSUB), :] += contrib

    @pl.when(jnp.logical_not(fits))
    def _():
        seg = lax.broadcasted_iota(jnp.int32, (S, R), 0)
        ohw = jnp.where(ids[None, :] == seg, e, 0.0).astype(jnp.bfloat16)
        contrib = jnp.dot(ohw, xb, preferred_element_type=jnp.float32)
        out_ref[...] += contrib

    @pl.when(i == G - 1)
    def _():
        out_ref[...] *= (1.0 / z_ref[0, 0])


def kernel(x, batch, W1, b1, W2, b2):
    del b2
    ids3 = batch.astype(jnp.int32).reshape(G, 1, R)
    b1r = b1.reshape(1, H)
    out = pl.pallas_call(
        _fused_kernel,
        grid=(G,),
        in_specs=[
            pl.BlockSpec((R, D), lambda i: (i, 0)),
            pl.BlockSpec((1, 1, R), lambda i: (i, 0, 0)),
            pl.BlockSpec((1, 1, R), lambda i: (i, 0, 0), memory_space=pltpu.SMEM),
            pl.BlockSpec((D, H), lambda i: (0, 0)),
            pl.BlockSpec((1, H), lambda i: (0, 0)),
            pl.BlockSpec((H, 1), lambda i: (0, 0)),
        ],
        out_specs=pl.BlockSpec((S, D), lambda i: (0, 0)),
        out_shape=jax.ShapeDtypeStruct((S, D), jnp.float32),
        scratch_shapes=[
            pltpu.SMEM((1, 1), jnp.float32),
            pltpu.SMEM((1, 1), jnp.float32),
        ],
    )(x, ids3, ids3, W1, b1r, W2)
    return out
